# baseline (speedup 1.0000x reference)
import functools

import jax
import jax.numpy as jnp
from jax import lax
from jax.experimental import pallas as pl
from jax.experimental.pallas import tpu as pltpu
from jax.experimental.pallas import tpu_sc as plsc

_B1, _B2, _V = 32, 8, 128256
_R = _B1 * _B2
_NW = 32
_K = 4
_R_SC = _NW * _K
_R_TC = _R - _R_SC
_ROWS_PER_W = _K
_L = 16
_U = 8
_NITER = 167
_CH = _U * _L * _NITER
_NC = _V // _CH
_NCHUNK = _ROWS_PER_W * _NC
_INT_MAX = 2**31 - 1

_mesh = plsc.VectorSubcoreMesh(
    core_axis_name="c", subcore_axis_name="s", num_cores=2, num_subcores=16)


def _merge(bv, bi, v, i):
    better = (v > bv) | ((v == bv) & (i < bi))
    return jnp.where(better, v, bv), jnp.where(better, i, bi)


def _scan_chunk(buf, chunk_base, iota):
    def body(i, carry):
        accs = list(carry)
        isplat = jnp.full((_L,), 0, jnp.int32) + i
        base = i * (_U * _L)
        for j in range(_U):
            vmax_j, vidx_j = accs[2 * j], accs[2 * j + 1]
            v = buf[pl.ds(base + j * _L, _L)]
            m = v > vmax_j
            accs[2 * j] = jnp.where(m, v, vmax_j)
            accs[2 * j + 1] = jnp.where(m, isplat, vidx_j)
        return tuple(accs)

    init = []
    for _ in range(_U):
        init.append(jnp.full((_L,), -jnp.inf, jnp.float32))
        init.append(jnp.zeros((_L,), jnp.int32))
    accs = lax.fori_loop(0, _NITER, body, tuple(init))

    bv = jnp.full((_L,), -jnp.inf, jnp.float32)
    bi = jnp.full((_L,), _INT_MAX, jnp.int32)
    for j in range(_U):
        vmax_j, vit_j = accs[2 * j], accs[2 * j + 1]
        gidx = vit_j * (_U * _L) + (chunk_base + j * _L) + iota
        bv, bi = _merge(bv, bi, vmax_j, gidx)
    return bv, bi


def _argmax_rows_body(x_hbm, out_hbm, buf0, buf1, res_v, sem0, sem1):
    wid = lax.axis_index("s") * 2 + lax.axis_index("c")
    iota = lax.iota(jnp.int32, _L)
    row0 = wid * _ROWS_PER_W

    def chunk_src(r, c):
        return x_hbm.at[row0 + r, pl.ds(pl.multiple_of(c * _CH, 8), _CH)]

    pltpu.async_copy(chunk_src(0, 0), buf0, sem0)

    def row_body(rl, res):
        def pair(ii, carry):
            rv, ri = carry
            c0 = ii * 2 * _CH
            pltpu.async_copy(chunk_src(rl, 2 * ii + 1), buf1, sem1)

            pltpu.make_async_copy(chunk_src(rl, 2 * ii), buf0, sem0).wait()
            cv, ci = _scan_chunk(buf0, c0, iota)
            rv, ri = _merge(rv, ri, cv, ci)

            nxt_r = jnp.where(ii < _NC // 2 - 1, rl, rl + 1)
            nxt_c = jnp.where(ii < _NC // 2 - 1, 2 * ii + 2, 0)

            @pl.when((rl < _ROWS_PER_W - 1) | (ii < _NC // 2 - 1))
            def _():
                pltpu.async_copy(chunk_src(nxt_r, nxt_c), buf0, sem0)

            pltpu.make_async_copy(chunk_src(rl, 2 * ii + 1), buf1, sem1).wait()
            cv, ci = _scan_chunk(buf1, c0 + _CH, iota)
            rv, ri = _merge(rv, ri, cv, ci)
            return rv, ri

        init = (
            jnp.full((_L,), -jnp.inf, jnp.float32),
            jnp.full((_L,), _INT_MAX, jnp.int32),
        )
        rv, ri = lax.fori_loop(0, _NC // 2, pair, init)
        for off in (8, 4, 2, 1):
            perm = iota ^ off
            v2 = rv.at[perm].get(mode="promise_in_bounds")
            i2 = ri.at[perm].get(mode="promise_in_bounds")
            rv, ri = _merge(rv, ri, v2, i2)
        return jnp.where(iota == rl, ri, res)

    res = lax.fori_loop(0, _ROWS_PER_W, row_body, jnp.zeros((_L,), jnp.int32))
    res_v[...] = res
    base_out = pl.multiple_of(wid * 8, 8)
    pltpu.sync_copy(res_v.at[pl.ds(0, 8)], out_hbm.at[pl.ds(base_out, 8)])


_argmax_rows = functools.partial(
    pl.kernel,
    mesh=_mesh,
    out_type=jax.ShapeDtypeStruct((_NW * 8,), jnp.int32),
    scratch_types=[
        pltpu.VMEM((_CH,), jnp.float32),
        pltpu.VMEM((_CH,), jnp.float32),
        pltpu.VMEM((_L,), jnp.int32),
        pltpu.SemaphoreType.DMA,
        pltpu.SemaphoreType.DMA,
    ],
)(_argmax_rows_body)


def _tc_body(x_ref, o_ref):
    x = x_ref[...]
    m = jnp.max(x, axis=1, keepdims=True)
    idx = lax.broadcasted_iota(jnp.int32, x.shape, 1)
    cand = jnp.where(x == m, idx, _INT_MAX)
    o_ref[...] = jnp.min(cand, axis=1, keepdims=True)


_tc_rows = 32


def _argmax_rows_tc(x):
    row_blk0 = _R_SC // _tc_rows
    return pl.pallas_call(
        _tc_body,
        grid=(_R_TC // _tc_rows,),
        in_specs=[pl.BlockSpec((_tc_rows, _V), lambda i: (i + row_blk0, 0))],
        out_specs=pl.BlockSpec((_tc_rows, 1), lambda i: (i, 0)),
        out_shape=jax.ShapeDtypeStruct((_R_TC, 1), jnp.int32),
    )(x)


def kernel(logits):
    flat = logits.reshape(_R, _V)
    sc_raw = _argmax_rows(flat)
    tc_out = _argmax_rows_tc(flat)
    sc_out = sc_raw.reshape(_NW, 8)[:, :_K].reshape(_R_SC)
    out = jnp.concatenate([sc_out, tc_out.reshape(_R_TC)])
    return out.reshape(_B1, _B2)


# speedup vs baseline: 1.0253x; 1.0253x over previous
import functools

import jax
import jax.numpy as jnp
from jax import lax
from jax.experimental import pallas as pl
from jax.experimental.pallas import tpu as pltpu
from jax.experimental.pallas import tpu_sc as plsc

_B1, _B2, _V = 32, 8, 128256
_R = _B1 * _B2
_NW = 32
_K = 4
_R_SC = _NW * _K
_R_TC = _R - _R_SC
_ROWS_PER_W = _K
_L = 16
_U = 8
_NITER = 167
_CH = _U * _L * _NITER
_NC = _V // _CH
_NCHUNK = _ROWS_PER_W * _NC
_INT_MAX = 2**31 - 1

_mesh = plsc.VectorSubcoreMesh(
    core_axis_name="c", subcore_axis_name="s", num_cores=2, num_subcores=16)


def _merge(bv, bi, v, i):
    better = (v > bv) | ((v == bv) & (i < bi))
    return jnp.where(better, v, bv), jnp.where(better, i, bi)


def _scan_chunk(buf, chunk_base, iota):
    def body(i, carry):
        accs = list(carry)
        isplat = jnp.full((_L,), 0, jnp.int32) + i
        base = i * (_U * _L)
        for j in range(_U):
            vmax_j, vidx_j = accs[2 * j], accs[2 * j + 1]
            v = buf[pl.ds(base + j * _L, _L)]
            m = v > vmax_j
            accs[2 * j] = jnp.where(m, v, vmax_j)
            accs[2 * j + 1] = jnp.where(m, isplat, vidx_j)
        return tuple(accs)

    init = []
    for _ in range(_U):
        init.append(jnp.full((_L,), -jnp.inf, jnp.float32))
        init.append(jnp.zeros((_L,), jnp.int32))
    accs = lax.fori_loop(0, _NITER, body, tuple(init))

    bv = jnp.full((_L,), -jnp.inf, jnp.float32)
    bi = jnp.full((_L,), _INT_MAX, jnp.int32)
    for j in range(_U):
        vmax_j, vit_j = accs[2 * j], accs[2 * j + 1]
        gidx = vit_j * (_U * _L) + (chunk_base + j * _L) + iota
        bv, bi = _merge(bv, bi, vmax_j, gidx)
    return bv, bi


def _argmax_rows_body(x_hbm, out_hbm, buf0, buf1, buf2, res_v,
                      sem0, sem1, sem2):
    wid = lax.axis_index("s") * 2 + lax.axis_index("c")
    iota = lax.iota(jnp.int32, _L)
    row0 = wid * _ROWS_PER_W

    def chunk_src(r, c):
        return x_hbm.at[row0 + r, pl.ds(pl.multiple_of(c * _CH, 8), _CH)]

    pltpu.async_copy(chunk_src(0, 0), buf0, sem0)
    pltpu.async_copy(chunk_src(0, 1), buf1, sem1)

    def row_body(rl, res):
        def triple(tt, carry):
            rv, ri = carry
            k = rl * _NC + 3 * tt
            c0 = 3 * tt * _CH
            legs = ((buf0, sem0, 0), (buf1, sem1, 1), (buf2, sem2, 2))
            for buf, sem, j in legs:
                cl = 3 * tt + j + 2
                nxt_r = jnp.where(cl < _NC, rl, rl + 1)
                nxt_c = jnp.where(cl < _NC, cl, cl - _NC)

                @pl.when(k + 2 + j < _NCHUNK)
                def _():
                    pltpu.async_copy(chunk_src(nxt_r, nxt_c),
                                     legs[(j + 2) % 3][0], legs[(j + 2) % 3][1])
                pltpu.make_async_copy(chunk_src(rl, 3 * tt + j), buf, sem).wait()
                cv, ci = _scan_chunk(buf, c0 + j * _CH, iota)
                rv, ri = _merge(rv, ri, cv, ci)
            return rv, ri

        init = (
            jnp.full((_L,), -jnp.inf, jnp.float32),
            jnp.full((_L,), _INT_MAX, jnp.int32),
        )
        rv, ri = lax.fori_loop(0, _NC // 3, triple, init)
        for off in (8, 4, 2, 1):
            perm = iota ^ off
            v2 = rv.at[perm].get(mode="promise_in_bounds")
            i2 = ri.at[perm].get(mode="promise_in_bounds")
            rv, ri = _merge(rv, ri, v2, i2)
        return jnp.where(iota == rl, ri, res)

    res = lax.fori_loop(0, _ROWS_PER_W, row_body, jnp.zeros((_L,), jnp.int32))
    res_v[...] = res
    base_out = pl.multiple_of(wid * 8, 8)
    pltpu.sync_copy(res_v.at[pl.ds(0, 8)], out_hbm.at[pl.ds(base_out, 8)])


_argmax_rows = functools.partial(
    pl.kernel,
    mesh=_mesh,
    out_type=jax.ShapeDtypeStruct((_NW * 8,), jnp.int32),
    scratch_types=[
        pltpu.VMEM((_CH,), jnp.float32),
        pltpu.VMEM((_CH,), jnp.float32),
        pltpu.VMEM((_CH,), jnp.float32),
        pltpu.VMEM((_L,), jnp.int32),
        pltpu.SemaphoreType.DMA,
        pltpu.SemaphoreType.DMA,
        pltpu.SemaphoreType.DMA,
    ],
)(_argmax_rows_body)


def _tc_body(x_ref, o_ref):
    x = x_ref[...]
    m = jnp.max(x, axis=1, keepdims=True)
    idx = lax.broadcasted_iota(jnp.int32, x.shape, 1)
    cand = jnp.where(x == m, idx, _INT_MAX)
    o_ref[...] = jnp.min(cand, axis=1, keepdims=True)


_tc_rows = 16


def _argmax_rows_tc(x):
    row_blk0 = _R_SC // _tc_rows
    return pl.pallas_call(
        _tc_body,
        grid=(_R_TC // _tc_rows,),
        in_specs=[pl.BlockSpec((_tc_rows, _V), lambda i: (i + row_blk0, 0))],
        out_specs=pl.BlockSpec((_tc_rows, 1), lambda i: (i, 0)),
        out_shape=jax.ShapeDtypeStruct((_R_TC, 1), jnp.int32),
    )(x)


def kernel(logits):
    flat = logits.reshape(_R, _V)
    sc_raw = _argmax_rows(flat)
    tc_out = _argmax_rows_tc(flat)
    sc_out = sc_raw.reshape(_NW, 8)[:, :_K].reshape(_R_SC)
    out = jnp.concatenate([sc_out, tc_out.reshape(_R_TC)])
    return out.reshape(_B1, _B2)
